# baseline (speedup 1.0000x reference)
import functools

import jax
import jax.numpy as jnp
from jax import lax
from jax.experimental import pallas as pl
from jax.experimental.pallas import tpu as pltpu
from jax.experimental.pallas import tpu_sc as plsc

N = 10000
E = 320000
D = 128

NC = 2
NS = 16
NW = NC * NS

NPAD = 10240
ROWS_PER_TILE = NPAD // NS
HR = NPAD // D
K = 80
EPT_AGG = E // NW
NB_AGG = EPT_AGG // K
CB = 16
NFULL = NB_AGG // CB
NTAIL = NB_AGG - NFULL * CB
EPT_DEG = E // NS

_mesh = plsc.VectorSubcoreMesh(core_axis_name="c", subcore_axis_name="s")


@functools.partial(
    pl.kernel,
    mesh=_mesh,
    out_type=jax.ShapeDtypeStruct((2, NPAD), jnp.float32),
    compiler_params=pltpu.CompilerParams(needs_layout_passes=False),
    scratch_types=[
        pltpu.VMEM((EPT_DEG,), jnp.int32),
        pltpu.VMEM((NPAD,), jnp.float32),
        pltpu.VMEM((ROWS_PER_TILE,), jnp.float32),
        pltpu.VMEM((ROWS_PER_TILE,), jnp.float32),
        pltpu.VMEM_SHARED((NS, NPAD), jnp.float32),
    ],
)
def _deg_kernel(eidx, out, idxv, hist, tmp, accb, stage):
    c = lax.axis_index("c")
    s = lax.axis_index("s")

    def zero_hist(i, carry):
        hist[pl.ds(i * 16, 16)] = jnp.zeros((16,), jnp.float32)
        return carry

    lax.fori_loop(0, NPAD // 16, zero_hist, 0)

    pltpu.sync_copy(eidx.at[c, s], idxv)

    ones16 = jnp.ones((16,), jnp.float32)

    def body(j, carry):
        iv = idxv[pl.ds(j * 16, 16)]
        plsc.addupdate_scatter(hist, [iv], ones16)
        return carry

    lax.fori_loop(0, EPT_DEG // 16, body, 0)

    pltpu.sync_copy(hist, stage.at[s])
    plsc.subcore_barrier()

    colbase = s * ROWS_PER_TILE
    pltpu.sync_copy(stage.at[0].at[pl.ds(colbase, ROWS_PER_TILE)], accb)

    def red(t, carry):
        pltpu.sync_copy(stage.at[t].at[pl.ds(colbase, ROWS_PER_TILE)], tmp)

        def addk(k, carry2):
            accb[pl.ds(k * 16, 16)] = (
                accb[pl.ds(k * 16, 16)] + tmp[pl.ds(k * 16, 16)]
            )
            return carry2

        lax.fori_loop(0, ROWS_PER_TILE // 16, addk, 0)
        return carry

    lax.fori_loop(1, NS, red, 0)
    pltpu.sync_copy(accb, out.at[c].at[pl.ds(colbase, ROWS_PER_TILE)])


@functools.partial(
    pl.kernel,
    mesh=_mesh,
    out_type=jax.ShapeDtypeStruct((2, NPAD, D), jnp.float32),
    scratch_types=[
        pltpu.VMEM((CB, K), jnp.int32),
        pltpu.VMEM((CB, K), jnp.int32),
        pltpu.VMEM((K, D), jnp.float32),
        pltpu.VMEM((K, D), jnp.float32),
        pltpu.VMEM((K, D), jnp.float32),
        pltpu.VMEM((K, D), jnp.float32),
        pltpu.VMEM_SHARED((NPAD, D), jnp.float32),
        pltpu.SemaphoreType.DMA,
        pltpu.SemaphoreType.DMA,
        pltpu.SemaphoreType.DMA,
        pltpu.SemaphoreType.DMA,
        pltpu.SemaphoreType.DMA,
        pltpu.SemaphoreType.DMA,
    ],
)
def _agg_kernel(g, src_r, dst_r, out, srcw, dstw, rows0, rows1, rows2,
                rows3, acc, gsem0, gsem1, ssem0, ssem1, ssem2, ssem3):
    c = lax.axis_index("c")
    s = lax.axis_index("s")
    wid = c * NS + s

    def fill_z(i, carry):
        for k in range(D // 16):
            rows0[i, pl.ds(k * 16, 16)] = jnp.zeros((16,), jnp.float32)
        return carry

    lax.fori_loop(0, K, fill_z, 0)

    base = s * ROWS_PER_TILE

    def zero_acc(t, carry):
        pltpu.sync_copy(rows0, acc.at[pl.ds(base + t * K, K)])
        return carry

    lax.fori_loop(0, ROWS_PER_TILE // K, zero_acc, 0)
    plsc.subcore_barrier()

    bufs = (rows0, rows1, rows2, rows3)
    gsems = (gsem0, gsem1)
    ssems = (ssem0, ssem1, ssem2, ssem3)

    def do_chunk(cb, nb):
        pltpu.sync_copy(src_r.at[wid].at[pl.ds(cb, nb)], srcw.at[pl.ds(0, nb)])
        pltpu.sync_copy(dst_r.at[wid].at[pl.ds(cb, nb)], dstw.at[pl.ds(0, nb)])

        gd = [None] * nb
        sd = [None] * nb

        def gather(b):
            gd[b] = pltpu.async_copy(
                g.at[srcw.at[b]], bufs[b % 4], gsems[b % 2])

        gather(0)
        if nb > 1:
            gather(1)
        for b in range(nb):
            gd[b].wait()
            sd[b] = pltpu.async_copy(
                bufs[b % 4], acc.at[dstw.at[b]], ssems[b % 4], add=True)
            if b + 2 < nb:
                if b - 2 >= 0:
                    sd[b - 2].wait()
                gather(b + 2)
        if nb >= 2:
            sd[nb - 2].wait()
        sd[nb - 1].wait()

    def chunk(cc, carry):
        do_chunk(cc * CB, CB)
        return carry

    lax.fori_loop(0, NFULL, chunk, 0)
    do_chunk(NFULL * CB, NTAIL)
    plsc.subcore_barrier()
    pltpu.sync_copy(
        acc.at[pl.ds(base, ROWS_PER_TILE)],
        out.at[c].at[pl.ds(base, ROWS_PER_TILE)],
    )


BLK = 2048
GRID = NPAD // BLK
DB = BLK // D


def _norm(dcol):
    return jnp.where(dcol > 0, lax.rsqrt(jnp.maximum(dcol, 1e-12)), 0.0)


def _tck1_body(deg_ref, x_ref, w_ref, o_ref):
    ns = _norm(deg_ref[0])
    o_ref[...] = jnp.dot(x_ref[...] * ns, w_ref[...],
                         preferred_element_type=jnp.float32)


def _tck1(deg, x, w):
    return pl.pallas_call(
        _tck1_body,
        grid=(GRID,),
        in_specs=[
            pl.BlockSpec((1, BLK, 1), lambda i: (0, i, 0)),
            pl.BlockSpec((BLK, D), lambda i: (i, 0)),
            pl.BlockSpec((D, D), lambda i: (0, 0)),
        ],
        out_specs=pl.BlockSpec((BLK, D), lambda i: (i, 0)),
        out_shape=jax.ShapeDtypeStruct((NPAD, D), jnp.float32),
    )(deg, x, w)


def _tck2_body(deg_ref, agg_ref, b_ref, w_ref, h1_ref, g2_ref):
    nd = _norm(deg_ref[1])
    ns = _norm(deg_ref[0])
    a = agg_ref[0] + agg_ref[1]
    h1 = jnp.maximum(a * nd + b_ref[...], 0.0)
    h1_ref[...] = h1
    g2_ref[...] = jnp.dot(h1 * ns, w_ref[...],
                          preferred_element_type=jnp.float32)


def _tck2(deg, agg, b, w):
    return pl.pallas_call(
        _tck2_body,
        grid=(GRID,),
        in_specs=[
            pl.BlockSpec((2, BLK, 1), lambda i: (0, i, 0)),
            pl.BlockSpec((2, BLK, D), lambda i: (0, i, 0)),
            pl.BlockSpec((1, D), lambda i: (0, 0)),
            pl.BlockSpec((D, D), lambda i: (0, 0)),
        ],
        out_specs=[
            pl.BlockSpec((BLK, D), lambda i: (i, 0)),
            pl.BlockSpec((BLK, D), lambda i: (i, 0)),
        ],
        out_shape=[
            jax.ShapeDtypeStruct((NPAD, D), jnp.float32),
            jax.ShapeDtypeStruct((NPAD, D), jnp.float32),
        ],
    )(deg, agg, b, w)


def _tck3_body(deg_ref, agg_ref, b_ref, o_ref):
    nd = _norm(deg_ref[1])
    a = agg_ref[0] + agg_ref[1]
    o_ref[...] = a * nd + b_ref[...]


def _tck3(deg, agg, b):
    return pl.pallas_call(
        _tck3_body,
        grid=(GRID,),
        in_specs=[
            pl.BlockSpec((2, BLK, 1), lambda i: (0, i, 0)),
            pl.BlockSpec((2, BLK, D), lambda i: (0, i, 0)),
            pl.BlockSpec((1, D), lambda i: (0, 0)),
        ],
        out_specs=pl.BlockSpec((BLK, D), lambda i: (i, 0)),
        out_shape=jax.ShapeDtypeStruct((NPAD, D), jnp.float32),
    )(deg, agg, b)


def kernel(x, edge_index, W1, b1, W2, b2):
    eidx_deg = edge_index.reshape(2, NS, EPT_DEG)
    src_r = edge_index[0].reshape(NW, NB_AGG, K)
    dst_r = edge_index[1].reshape(NW, NB_AGG, K)
    xp = jnp.pad(x, ((0, NPAD - N), (0, 0)))

    deg = _deg_kernel(eidx_deg)[:, :, None]
    g1 = _tck1(deg, xp, W1)
    agg1 = _agg_kernel(g1, src_r, dst_r)
    h1, g2 = _tck2(deg, agg1, b1.reshape(1, D), W2)
    agg2 = _agg_kernel(g2, src_r, dst_r)
    h2 = _tck3(deg, agg2, b2.reshape(1, D))
    return (h2[:N], h1[:N])


# speedup vs baseline: 1.0770x; 1.0770x over previous
import functools

import jax
import jax.numpy as jnp
from jax import lax
from jax.experimental import pallas as pl
from jax.experimental.pallas import tpu as pltpu
from jax.experimental.pallas import tpu_sc as plsc

N = 10000
E = 320000
D = 128

NC = 2
NS = 16
NW = NC * NS

NPAD = 10240
ROWS_PER_TILE = NPAD // NS
HR = NPAD // D
K = 80
EPT_AGG = E // NW
NB_AGG = EPT_AGG // K
CB = 16
NFULL = NB_AGG // CB
NTAIL = NB_AGG - NFULL * CB
EPT_DEG = E // NS

_mesh = plsc.VectorSubcoreMesh(core_axis_name="c", subcore_axis_name="s")


@functools.partial(
    pl.kernel,
    mesh=_mesh,
    out_type=jax.ShapeDtypeStruct((2, NPAD), jnp.float32),
    compiler_params=pltpu.CompilerParams(needs_layout_passes=False),
    scratch_types=[
        pltpu.VMEM((EPT_DEG,), jnp.int32),
        pltpu.VMEM((NPAD,), jnp.float32),
        pltpu.VMEM((ROWS_PER_TILE,), jnp.float32),
        pltpu.VMEM((ROWS_PER_TILE,), jnp.float32),
        pltpu.VMEM_SHARED((NS, NPAD), jnp.float32),
    ],
)
def _deg_kernel(eidx, out, idxv, hist, tmp, accb, stage):
    c = lax.axis_index("c")
    s = lax.axis_index("s")

    def zero_hist(i, carry):
        hist[pl.ds(i * 16, 16)] = jnp.zeros((16,), jnp.float32)
        return carry

    lax.fori_loop(0, NPAD // 16, zero_hist, 0)

    pltpu.sync_copy(eidx.at[c, s], idxv)

    ones16 = jnp.ones((16,), jnp.float32)

    def body(j, carry):
        iv = idxv[pl.ds(j * 16, 16)]
        plsc.addupdate_scatter(hist, [iv], ones16)
        return carry

    lax.fori_loop(0, EPT_DEG // 16, body, 0)

    pltpu.sync_copy(hist, stage.at[s])
    plsc.subcore_barrier()

    colbase = s * ROWS_PER_TILE
    pltpu.sync_copy(stage.at[0].at[pl.ds(colbase, ROWS_PER_TILE)], accb)

    def red(t, carry):
        pltpu.sync_copy(stage.at[t].at[pl.ds(colbase, ROWS_PER_TILE)], tmp)

        def addk(k, carry2):
            accb[pl.ds(k * 16, 16)] = (
                accb[pl.ds(k * 16, 16)] + tmp[pl.ds(k * 16, 16)]
            )
            return carry2

        lax.fori_loop(0, ROWS_PER_TILE // 16, addk, 0)
        return carry

    lax.fori_loop(1, NS, red, 0)
    pltpu.sync_copy(accb, out.at[c].at[pl.ds(colbase, ROWS_PER_TILE)])


@functools.partial(
    pl.kernel,
    mesh=_mesh,
    out_type=jax.ShapeDtypeStruct((2, NPAD, D), jnp.float32),
    scratch_types=[
        pltpu.VMEM((CB, K), jnp.int32),
        pltpu.VMEM((CB, K), jnp.int32),
        pltpu.VMEM((K, D), jnp.float32),
        pltpu.VMEM((K, D), jnp.float32),
        pltpu.VMEM((K, D), jnp.float32),
        pltpu.VMEM((K, D), jnp.float32),
        pltpu.VMEM_SHARED((NPAD, D), jnp.float32),
        pltpu.SemaphoreType.DMA,
        pltpu.SemaphoreType.DMA,
        pltpu.SemaphoreType.DMA,
        pltpu.SemaphoreType.DMA,
    ],
)
def _agg_kernel(g, src_r, dst_r, out, srcw, dstw, rows0, rows1, rows2,
                rows3, acc, gsem0, gsem1, ssem0, ssem1):
    c = lax.axis_index("c")
    s = lax.axis_index("s")
    wid = c * NS + s

    def fill_z(i, carry):
        for k in range(D // 16):
            rows0[i, pl.ds(k * 16, 16)] = jnp.zeros((16,), jnp.float32)
        return carry

    lax.fori_loop(0, K, fill_z, 0)

    base = s * ROWS_PER_TILE

    def zero_acc(t, carry):
        pltpu.sync_copy(rows0, acc.at[pl.ds(base + t * K, K)])
        return carry

    lax.fori_loop(0, ROWS_PER_TILE // K, zero_acc, 0)
    plsc.subcore_barrier()

    bufs = (rows0, rows1, rows2, rows3)
    gsems = (gsem0, gsem1, ssem0, ssem1)

    def do_chunk(cb, nb):
        pltpu.sync_copy(src_r.at[wid].at[pl.ds(cb, nb)], srcw.at[pl.ds(0, nb)])
        pltpu.sync_copy(dst_r.at[wid].at[pl.ds(cb, nb)], dstw.at[pl.ds(0, nb)])

        gd = [None] * nb

        def gather(b):
            gd[b] = pltpu.async_copy(
                g.at[srcw.at[b]], bufs[b % 4], gsems[b % 4])

        for p in range(min(3, nb)):
            gather(p)
        for b in range(nb):
            gd[b].wait()
            pltpu.sync_copy(bufs[b % 4], acc.at[dstw.at[b]], add=True)
            if b + 3 < nb:
                gather(b + 3)

    def chunk(cc, carry):
        do_chunk(cc * CB, CB)
        return carry

    lax.fori_loop(0, NFULL, chunk, 0)
    do_chunk(NFULL * CB, NTAIL)
    plsc.subcore_barrier()
    pltpu.sync_copy(
        acc.at[pl.ds(base, ROWS_PER_TILE)],
        out.at[c].at[pl.ds(base, ROWS_PER_TILE)],
    )


BLK = 2048
GRID = NPAD // BLK
DB = BLK // D


def _norm(dcol):
    return jnp.where(dcol > 0, lax.rsqrt(jnp.maximum(dcol, 1e-12)), 0.0)


def _tck1_body(deg_ref, x_ref, w_ref, o_ref):
    ns = _norm(deg_ref[0])
    o_ref[...] = jnp.dot(x_ref[...] * ns, w_ref[...],
                         preferred_element_type=jnp.float32)


def _tck1(deg, x, w):
    return pl.pallas_call(
        _tck1_body,
        grid=(GRID,),
        in_specs=[
            pl.BlockSpec((1, BLK, 1), lambda i: (0, i, 0)),
            pl.BlockSpec((BLK, D), lambda i: (i, 0)),
            pl.BlockSpec((D, D), lambda i: (0, 0)),
        ],
        out_specs=pl.BlockSpec((BLK, D), lambda i: (i, 0)),
        out_shape=jax.ShapeDtypeStruct((NPAD, D), jnp.float32),
    )(deg, x, w)


def _tck2_body(deg_ref, agg_ref, b_ref, w_ref, h1_ref, g2_ref):
    nd = _norm(deg_ref[1])
    ns = _norm(deg_ref[0])
    a = agg_ref[0] + agg_ref[1]
    h1 = jnp.maximum(a * nd + b_ref[...], 0.0)
    h1_ref[...] = h1
    g2_ref[...] = jnp.dot(h1 * ns, w_ref[...],
                          preferred_element_type=jnp.float32)


def _tck2(deg, agg, b, w):
    return pl.pallas_call(
        _tck2_body,
        grid=(GRID,),
        in_specs=[
            pl.BlockSpec((2, BLK, 1), lambda i: (0, i, 0)),
            pl.BlockSpec((2, BLK, D), lambda i: (0, i, 0)),
            pl.BlockSpec((1, D), lambda i: (0, 0)),
            pl.BlockSpec((D, D), lambda i: (0, 0)),
        ],
        out_specs=[
            pl.BlockSpec((BLK, D), lambda i: (i, 0)),
            pl.BlockSpec((BLK, D), lambda i: (i, 0)),
        ],
        out_shape=[
            jax.ShapeDtypeStruct((NPAD, D), jnp.float32),
            jax.ShapeDtypeStruct((NPAD, D), jnp.float32),
        ],
    )(deg, agg, b, w)


def _tck3_body(deg_ref, agg_ref, b_ref, o_ref):
    nd = _norm(deg_ref[1])
    a = agg_ref[0] + agg_ref[1]
    o_ref[...] = a * nd + b_ref[...]


def _tck3(deg, agg, b):
    return pl.pallas_call(
        _tck3_body,
        grid=(GRID,),
        in_specs=[
            pl.BlockSpec((2, BLK, 1), lambda i: (0, i, 0)),
            pl.BlockSpec((2, BLK, D), lambda i: (0, i, 0)),
            pl.BlockSpec((1, D), lambda i: (0, 0)),
        ],
        out_specs=pl.BlockSpec((BLK, D), lambda i: (i, 0)),
        out_shape=jax.ShapeDtypeStruct((NPAD, D), jnp.float32),
    )(deg, agg, b)


def kernel(x, edge_index, W1, b1, W2, b2):
    eidx_deg = edge_index.reshape(2, NS, EPT_DEG)
    src_r = edge_index[0].reshape(NW, NB_AGG, K)
    dst_r = edge_index[1].reshape(NW, NB_AGG, K)
    xp = jnp.pad(x, ((0, NPAD - N), (0, 0)))

    deg = _deg_kernel(eidx_deg)[:, :, None]
    g1 = _tck1(deg, xp, W1)
    agg1 = _agg_kernel(g1, src_r, dst_r)
    h1, g2 = _tck2(deg, agg1, b1.reshape(1, D), W2)
    agg2 = _agg_kernel(g2, src_r, dst_r)
    h2 = _tck3(deg, agg2, b2.reshape(1, D))
    return (h2[:N], h1[:N])


# speedup vs baseline: 1.1151x; 1.0354x over previous
import functools

import jax
import jax.numpy as jnp
from jax import lax
from jax.experimental import pallas as pl
from jax.experimental.pallas import tpu as pltpu
from jax.experimental.pallas import tpu_sc as plsc

N = 10000
E = 320000
D = 128

NC = 2
NS = 16
NW = NC * NS

NPAD = 10240
ROWS_PER_TILE = NPAD // NS
HR = NPAD // D
K = 80
EPT_AGG = E // NW
NB_AGG = EPT_AGG // K
CB = 24
NFULL = NB_AGG // CB
NTAIL = NB_AGG - NFULL * CB
EPT_DEG = E // NS

_mesh = plsc.VectorSubcoreMesh(core_axis_name="c", subcore_axis_name="s")


@functools.partial(
    pl.kernel,
    mesh=_mesh,
    out_type=jax.ShapeDtypeStruct((2, NPAD), jnp.float32),
    compiler_params=pltpu.CompilerParams(needs_layout_passes=False),
    scratch_types=[
        pltpu.VMEM((EPT_DEG,), jnp.int32),
        pltpu.VMEM((NPAD,), jnp.float32),
        pltpu.VMEM((ROWS_PER_TILE,), jnp.float32),
        pltpu.VMEM((ROWS_PER_TILE,), jnp.float32),
        pltpu.VMEM_SHARED((NS, NPAD), jnp.float32),
    ],
)
def _deg_kernel(eidx, out, idxv, hist, tmp, accb, stage):
    c = lax.axis_index("c")
    s = lax.axis_index("s")

    def zero_hist(i, carry):
        hist[pl.ds(i * 16, 16)] = jnp.zeros((16,), jnp.float32)
        return carry

    lax.fori_loop(0, NPAD // 16, zero_hist, 0)

    pltpu.sync_copy(eidx.at[c, s], idxv)

    ones16 = jnp.ones((16,), jnp.float32)

    def body(j, carry):
        iv = idxv[pl.ds(j * 16, 16)]
        plsc.addupdate_scatter(hist, [iv], ones16)
        return carry

    lax.fori_loop(0, EPT_DEG // 16, body, 0)

    pltpu.sync_copy(hist, stage.at[s])
    plsc.subcore_barrier()

    colbase = s * ROWS_PER_TILE
    pltpu.sync_copy(stage.at[0].at[pl.ds(colbase, ROWS_PER_TILE)], accb)

    def red(t, carry):
        pltpu.sync_copy(stage.at[t].at[pl.ds(colbase, ROWS_PER_TILE)], tmp)

        def addk(k, carry2):
            accb[pl.ds(k * 16, 16)] = (
                accb[pl.ds(k * 16, 16)] + tmp[pl.ds(k * 16, 16)]
            )
            return carry2

        lax.fori_loop(0, ROWS_PER_TILE // 16, addk, 0)
        return carry

    lax.fori_loop(1, NS, red, 0)
    pltpu.sync_copy(accb, out.at[c].at[pl.ds(colbase, ROWS_PER_TILE)])


@functools.partial(
    pl.kernel,
    mesh=_mesh,
    out_type=jax.ShapeDtypeStruct((2, NPAD, D), jnp.float32),
    scratch_types=[
        pltpu.VMEM((CB, K), jnp.int32),
        pltpu.VMEM((CB, K), jnp.int32),
        pltpu.VMEM((K, D), jnp.float32),
        pltpu.VMEM((K, D), jnp.float32),
        pltpu.VMEM((K, D), jnp.float32),
        pltpu.VMEM((K, D), jnp.float32),
        pltpu.VMEM_SHARED((NPAD, D), jnp.float32),
        pltpu.SemaphoreType.DMA,
        pltpu.SemaphoreType.DMA,
        pltpu.SemaphoreType.DMA,
        pltpu.SemaphoreType.DMA,
    ],
)
def _agg_kernel(g, src_r, dst_r, out, srcw, dstw, rows0, rows1, rows2,
                rows3, acc, gsem0, gsem1, ssem0, ssem1):
    c = lax.axis_index("c")
    s = lax.axis_index("s")
    wid = c * NS + s

    def fill_z(i, carry):
        for k in range(D // 16):
            rows0[i, pl.ds(k * 16, 16)] = jnp.zeros((16,), jnp.float32)
        return carry

    lax.fori_loop(0, K, fill_z, 0)

    base = s * ROWS_PER_TILE

    def zero_acc(t, carry):
        pltpu.sync_copy(rows0, acc.at[pl.ds(base + t * K, K)])
        return carry

    lax.fori_loop(0, ROWS_PER_TILE // K, zero_acc, 0)
    plsc.subcore_barrier()

    bufs = (rows0, rows1, rows2, rows3)
    gsems = (gsem0, gsem1, ssem0, ssem1)

    def do_chunk(cb, nb):
        pltpu.sync_copy(src_r.at[wid].at[pl.ds(cb, nb)], srcw.at[pl.ds(0, nb)])
        pltpu.sync_copy(dst_r.at[wid].at[pl.ds(cb, nb)], dstw.at[pl.ds(0, nb)])

        gd = [None] * nb

        def gather(b):
            gd[b] = pltpu.async_copy(
                g.at[srcw.at[b]], bufs[b % 4], gsems[b % 4])

        for p in range(min(3, nb)):
            gather(p)
        for b in range(nb):
            gd[b].wait()
            pltpu.sync_copy(bufs[b % 4], acc.at[dstw.at[b]], add=True)
            if b + 3 < nb:
                gather(b + 3)

    def chunk(cc, carry):
        do_chunk(cc * CB, CB)
        return carry

    lax.fori_loop(0, NFULL, chunk, 0)
    do_chunk(NFULL * CB, NTAIL)
    plsc.subcore_barrier()
    pltpu.sync_copy(
        acc.at[pl.ds(base, ROWS_PER_TILE)],
        out.at[c].at[pl.ds(base, ROWS_PER_TILE)],
    )


BLK = 2048
GRID = NPAD // BLK
DB = BLK // D


def _norm(dcol):
    return jnp.where(dcol > 0, lax.rsqrt(jnp.maximum(dcol, 1e-12)), 0.0)


def _tck1_body(deg_ref, x_ref, w_ref, o_ref):
    ns = _norm(deg_ref[0])
    o_ref[...] = jnp.dot(x_ref[...] * ns, w_ref[...],
                         preferred_element_type=jnp.float32)


def _tck1(deg, x, w):
    return pl.pallas_call(
        _tck1_body,
        grid=(GRID,),
        in_specs=[
            pl.BlockSpec((1, BLK, 1), lambda i: (0, i, 0)),
            pl.BlockSpec((BLK, D), lambda i: (i, 0)),
            pl.BlockSpec((D, D), lambda i: (0, 0)),
        ],
        out_specs=pl.BlockSpec((BLK, D), lambda i: (i, 0)),
        out_shape=jax.ShapeDtypeStruct((NPAD, D), jnp.float32),
    )(deg, x, w)


def _tck2_body(deg_ref, agg_ref, b_ref, w_ref, h1_ref, g2_ref):
    nd = _norm(deg_ref[1])
    ns = _norm(deg_ref[0])
    a = agg_ref[0] + agg_ref[1]
    h1 = jnp.maximum(a * nd + b_ref[...], 0.0)
    h1_ref[...] = h1
    g2_ref[...] = jnp.dot(h1 * ns, w_ref[...],
                          preferred_element_type=jnp.float32)


def _tck2(deg, agg, b, w):
    return pl.pallas_call(
        _tck2_body,
        grid=(GRID,),
        in_specs=[
            pl.BlockSpec((2, BLK, 1), lambda i: (0, i, 0)),
            pl.BlockSpec((2, BLK, D), lambda i: (0, i, 0)),
            pl.BlockSpec((1, D), lambda i: (0, 0)),
            pl.BlockSpec((D, D), lambda i: (0, 0)),
        ],
        out_specs=[
            pl.BlockSpec((BLK, D), lambda i: (i, 0)),
            pl.BlockSpec((BLK, D), lambda i: (i, 0)),
        ],
        out_shape=[
            jax.ShapeDtypeStruct((NPAD, D), jnp.float32),
            jax.ShapeDtypeStruct((NPAD, D), jnp.float32),
        ],
    )(deg, agg, b, w)


def _tck3_body(deg_ref, agg_ref, b_ref, o_ref):
    nd = _norm(deg_ref[1])
    a = agg_ref[0] + agg_ref[1]
    o_ref[...] = a * nd + b_ref[...]


def _tck3(deg, agg, b):
    return pl.pallas_call(
        _tck3_body,
        grid=(GRID,),
        in_specs=[
            pl.BlockSpec((2, BLK, 1), lambda i: (0, i, 0)),
            pl.BlockSpec((2, BLK, D), lambda i: (0, i, 0)),
            pl.BlockSpec((1, D), lambda i: (0, 0)),
        ],
        out_specs=pl.BlockSpec((BLK, D), lambda i: (i, 0)),
        out_shape=jax.ShapeDtypeStruct((NPAD, D), jnp.float32),
    )(deg, agg, b)


def kernel(x, edge_index, W1, b1, W2, b2):
    eidx_deg = edge_index.reshape(2, NS, EPT_DEG)
    src_r = edge_index[0].reshape(NW, NB_AGG, K)
    dst_r = edge_index[1].reshape(NW, NB_AGG, K)
    xp = jnp.pad(x, ((0, NPAD - N), (0, 0)))

    deg = _deg_kernel(eidx_deg)[:, :, None]
    g1 = _tck1(deg, xp, W1)
    agg1 = _agg_kernel(g1, src_r, dst_r)
    h1, g2 = _tck2(deg, agg1, b1.reshape(1, D), W2)
    agg2 = _agg_kernel(g2, src_r, dst_r)
    h2 = _tck3(deg, agg2, b2.reshape(1, D))
    return (h2[:N], h1[:N])


# speedup vs baseline: 1.1478x; 1.0293x over previous
import functools

import jax
import jax.numpy as jnp
from jax import lax
from jax.experimental import pallas as pl
from jax.experimental.pallas import tpu as pltpu
from jax.experimental.pallas import tpu_sc as plsc

N = 10000
E = 320000
D = 128

NC = 2
NS = 16
NW = NC * NS

NPAD = 10240
ROWS_PER_TILE = NPAD // NS
HR = NPAD // D
K = 80
EPT_AGG = E // NW
NB_AGG = EPT_AGG // K
CB = 24
NFULL = NB_AGG // CB
NTAIL = NB_AGG - NFULL * CB
EPT_DEG = E // NS

_mesh = plsc.VectorSubcoreMesh(core_axis_name="c", subcore_axis_name="s")


@functools.partial(
    pl.kernel,
    mesh=_mesh,
    out_type=jax.ShapeDtypeStruct((2, NPAD), jnp.float32),
    compiler_params=pltpu.CompilerParams(needs_layout_passes=False),
    scratch_types=[
        pltpu.VMEM((EPT_DEG,), jnp.int32),
        pltpu.VMEM((NPAD,), jnp.float32),
        pltpu.VMEM((ROWS_PER_TILE,), jnp.float32),
        pltpu.VMEM((ROWS_PER_TILE,), jnp.float32),
        pltpu.VMEM_SHARED((NS, NPAD), jnp.float32),
    ],
)
def _deg_kernel(eidx, out, idxv, hist, tmp, accb, stage):
    c = lax.axis_index("c")
    s = lax.axis_index("s")

    def zero_hist(i, carry):
        hist[pl.ds(i * 16, 16)] = jnp.zeros((16,), jnp.float32)
        return carry

    lax.fori_loop(0, NPAD // 16, zero_hist, 0)

    pltpu.sync_copy(eidx.at[c, s], idxv)

    ones16 = jnp.ones((16,), jnp.float32)

    def body(j, carry):
        iv = idxv[pl.ds(j * 16, 16)]
        plsc.addupdate_scatter(hist, [iv], ones16)
        return carry

    lax.fori_loop(0, EPT_DEG // 16, body, 0)

    pltpu.sync_copy(hist, stage.at[s])
    plsc.subcore_barrier()

    colbase = s * ROWS_PER_TILE
    pltpu.sync_copy(stage.at[0].at[pl.ds(colbase, ROWS_PER_TILE)], accb)

    def red(t, carry):
        pltpu.sync_copy(stage.at[t].at[pl.ds(colbase, ROWS_PER_TILE)], tmp)

        def addk(k, carry2):
            accb[pl.ds(k * 16, 16)] = (
                accb[pl.ds(k * 16, 16)] + tmp[pl.ds(k * 16, 16)]
            )
            return carry2

        lax.fori_loop(0, ROWS_PER_TILE // 16, addk, 0)
        return carry

    lax.fori_loop(1, NS, red, 0)
    pltpu.sync_copy(accb, out.at[c].at[pl.ds(colbase, ROWS_PER_TILE)])


@functools.partial(
    pl.kernel,
    mesh=_mesh,
    out_type=jax.ShapeDtypeStruct((2, NPAD, D), jnp.float32),
    scratch_types=[
        pltpu.VMEM((CB, K), jnp.int32),
        pltpu.VMEM((CB, K), jnp.int32),
        pltpu.VMEM((K, D), jnp.float32),
        pltpu.VMEM((K, D), jnp.float32),
        pltpu.VMEM((K, D), jnp.float32),
        pltpu.VMEM((K, D), jnp.float32),
        pltpu.VMEM_SHARED((NPAD, D), jnp.float32),
        pltpu.SemaphoreType.DMA,
        pltpu.SemaphoreType.DMA,
        pltpu.SemaphoreType.DMA,
        pltpu.SemaphoreType.DMA,
    ],
)
def _agg_kernel(g, src_r, dst_r, out, srcw, dstw, rows0, rows1, rows2,
                rows3, acc, gsem0, gsem1, ssem0, ssem1):
    c = lax.axis_index("c")
    s = lax.axis_index("s")
    wid = c * NS + s

    def fill_z(i, carry):
        for k in range(D // 16):
            rows0[i, pl.ds(k * 16, 16)] = jnp.zeros((16,), jnp.float32)
        return carry

    lax.fori_loop(0, K, fill_z, 0)

    base = s * ROWS_PER_TILE

    def zero_acc(t, carry):
        pltpu.sync_copy(rows0, acc.at[pl.ds(base + t * K, K)])
        return carry

    lax.fori_loop(0, ROWS_PER_TILE // K, zero_acc, 0)
    plsc.subcore_barrier()

    bufs = (rows0, rows1, rows2, rows3)
    gsems = (gsem0, gsem1, ssem0, ssem1)

    def do_chunk(cb, nb):
        pltpu.sync_copy(src_r.at[wid].at[pl.ds(cb, nb)], srcw.at[pl.ds(0, nb)])
        pltpu.sync_copy(dst_r.at[wid].at[pl.ds(cb, nb)], dstw.at[pl.ds(0, nb)])

        gd = [None] * nb

        def gather(b):
            gd[b] = pltpu.async_copy(
                g.at[srcw.at[b]], bufs[b % 4], gsems[b % 4])

        for p in range(min(3, nb)):
            gather(p)
        for b in range(nb):
            gd[b].wait()
            pltpu.sync_copy(bufs[b % 4], acc.at[dstw.at[b]], add=True)
            if b + 3 < nb:
                gather(b + 3)

    def chunk(cc, carry):
        do_chunk(cc * CB, CB)
        return carry

    lax.fori_loop(0, NFULL, chunk, 0)
    do_chunk(NFULL * CB, NTAIL)
    plsc.subcore_barrier()
    pltpu.sync_copy(
        acc.at[pl.ds(base, ROWS_PER_TILE)],
        out.at[c].at[pl.ds(base, ROWS_PER_TILE)],
    )


BLK = 2000
GRID = N // BLK


def _norm(dcol):
    return jnp.where(dcol > 0, lax.rsqrt(jnp.maximum(dcol, 1e-12)), 0.0)


def _tck1_body(deg_ref, x_ref, w_ref, o_ref):
    ns = _norm(deg_ref[0])
    o_ref[...] = jnp.dot(x_ref[...] * ns, w_ref[...],
                         preferred_element_type=jnp.float32)


def _tck1(deg, x, w):
    return pl.pallas_call(
        _tck1_body,
        grid=(GRID,),
        in_specs=[
            pl.BlockSpec((1, BLK, 1), lambda i: (0, i, 0)),
            pl.BlockSpec((BLK, D), lambda i: (i, 0)),
            pl.BlockSpec((D, D), lambda i: (0, 0)),
        ],
        out_specs=pl.BlockSpec((BLK, D), lambda i: (i, 0)),
        out_shape=jax.ShapeDtypeStruct((N, D), jnp.float32),
    )(deg, x, w)


def _tck2_body(deg_ref, agg_ref, b_ref, w_ref, h1_ref, g2_ref):
    nd = _norm(deg_ref[1])
    ns = _norm(deg_ref[0])
    a = agg_ref[0] + agg_ref[1]
    h1 = jnp.maximum(a * nd + b_ref[...], 0.0)
    h1_ref[...] = h1
    g2_ref[...] = jnp.dot(h1 * ns, w_ref[...],
                          preferred_element_type=jnp.float32)


def _tck2(deg, agg, b, w):
    return pl.pallas_call(
        _tck2_body,
        grid=(GRID,),
        in_specs=[
            pl.BlockSpec((2, BLK, 1), lambda i: (0, i, 0)),
            pl.BlockSpec((2, BLK, D), lambda i: (0, i, 0)),
            pl.BlockSpec((1, D), lambda i: (0, 0)),
            pl.BlockSpec((D, D), lambda i: (0, 0)),
        ],
        out_specs=[
            pl.BlockSpec((BLK, D), lambda i: (i, 0)),
            pl.BlockSpec((BLK, D), lambda i: (i, 0)),
        ],
        out_shape=[
            jax.ShapeDtypeStruct((N, D), jnp.float32),
            jax.ShapeDtypeStruct((N, D), jnp.float32),
        ],
    )(deg, agg, b, w)


def _tck3_body(deg_ref, agg_ref, b_ref, o_ref):
    nd = _norm(deg_ref[1])
    a = agg_ref[0] + agg_ref[1]
    o_ref[...] = a * nd + b_ref[...]


def _tck3(deg, agg, b):
    return pl.pallas_call(
        _tck3_body,
        grid=(GRID,),
        in_specs=[
            pl.BlockSpec((2, BLK, 1), lambda i: (0, i, 0)),
            pl.BlockSpec((2, BLK, D), lambda i: (0, i, 0)),
            pl.BlockSpec((1, D), lambda i: (0, 0)),
        ],
        out_specs=pl.BlockSpec((BLK, D), lambda i: (i, 0)),
        out_shape=jax.ShapeDtypeStruct((N, D), jnp.float32),
    )(deg, agg, b)


def kernel(x, edge_index, W1, b1, W2, b2):
    eidx_deg = edge_index.reshape(2, NS, EPT_DEG)
    src_r = edge_index[0].reshape(NW, NB_AGG, K)
    dst_r = edge_index[1].reshape(NW, NB_AGG, K)
    deg = _deg_kernel(eidx_deg)[:, :, None]
    g1 = _tck1(deg, x, W1)
    agg1 = _agg_kernel(g1, src_r, dst_r)
    h1, g2 = _tck2(deg, agg1, b1.reshape(1, D), W2)
    agg2 = _agg_kernel(g2, src_r, dst_r)
    h2 = _tck3(deg, agg2, b2.reshape(1, D))
    return (h2, h1)


# speedup vs baseline: 1.1505x; 1.0023x over previous
import functools

import jax
import jax.numpy as jnp
from jax import lax
from jax.experimental import pallas as pl
from jax.experimental.pallas import tpu as pltpu
from jax.experimental.pallas import tpu_sc as plsc

N = 10000
E = 320000
D = 128

NC = 2
NS = 16
NW = NC * NS

NPAD = 10240
ROWS_PER_TILE = NPAD // NS
HR = NPAD // D
K = 80
EPT_AGG = E // NW
NB_AGG = EPT_AGG // K
CB = 24
NFULL = NB_AGG // CB
NTAIL = NB_AGG - NFULL * CB
EPT_DEG = E // NS

_mesh = plsc.VectorSubcoreMesh(core_axis_name="c", subcore_axis_name="s")


@functools.partial(
    pl.kernel,
    mesh=_mesh,
    out_type=jax.ShapeDtypeStruct((2, NPAD), jnp.float32),
    compiler_params=pltpu.CompilerParams(needs_layout_passes=False),
    scratch_types=[
        pltpu.VMEM((EPT_DEG,), jnp.int32),
        pltpu.VMEM((NPAD,), jnp.float32),
        pltpu.VMEM((ROWS_PER_TILE,), jnp.float32),
        pltpu.VMEM((ROWS_PER_TILE,), jnp.float32),
        pltpu.VMEM_SHARED((NS, NPAD), jnp.float32),
    ],
)
def _deg_kernel(eidx, out, idxv, hist, tmp, accb, stage):
    c = lax.axis_index("c")
    s = lax.axis_index("s")

    def zero_hist(i, carry):
        hist[pl.ds(i * 16, 16)] = jnp.zeros((16,), jnp.float32)
        return carry

    lax.fori_loop(0, NPAD // 16, zero_hist, 0)

    pltpu.sync_copy(eidx.at[c, s], idxv)

    ones16 = jnp.ones((16,), jnp.float32)

    def body(j, carry):
        for u in range(8):
            iv = idxv[pl.ds(j * 128 + u * 16, 16)]
            plsc.addupdate_scatter(hist, [iv], ones16)
        return carry

    lax.fori_loop(0, EPT_DEG // 128, body, 0)

    pltpu.sync_copy(hist, stage.at[s])
    plsc.subcore_barrier()

    colbase = s * ROWS_PER_TILE
    pltpu.sync_copy(stage.at[0].at[pl.ds(colbase, ROWS_PER_TILE)], accb)

    def red(t, carry):
        pltpu.sync_copy(stage.at[t].at[pl.ds(colbase, ROWS_PER_TILE)], tmp)

        def addk(k, carry2):
            accb[pl.ds(k * 16, 16)] = (
                accb[pl.ds(k * 16, 16)] + tmp[pl.ds(k * 16, 16)]
            )
            return carry2

        lax.fori_loop(0, ROWS_PER_TILE // 16, addk, 0)
        return carry

    lax.fori_loop(1, NS, red, 0)
    pltpu.sync_copy(accb, out.at[c].at[pl.ds(colbase, ROWS_PER_TILE)])


@functools.partial(
    pl.kernel,
    mesh=_mesh,
    out_type=jax.ShapeDtypeStruct((2, NPAD, D), jnp.float32),
    scratch_types=[
        pltpu.VMEM((CB, K), jnp.int32),
        pltpu.VMEM((CB, K), jnp.int32),
        pltpu.VMEM((K, D), jnp.float32),
        pltpu.VMEM((K, D), jnp.float32),
        pltpu.VMEM((K, D), jnp.float32),
        pltpu.VMEM((K, D), jnp.float32),
        pltpu.VMEM_SHARED((NPAD, D), jnp.float32),
        pltpu.SemaphoreType.DMA,
        pltpu.SemaphoreType.DMA,
        pltpu.SemaphoreType.DMA,
        pltpu.SemaphoreType.DMA,
    ],
)
def _agg_kernel(g, src_r, dst_r, out, srcw, dstw, rows0, rows1, rows2,
                rows3, acc, gsem0, gsem1, ssem0, ssem1):
    c = lax.axis_index("c")
    s = lax.axis_index("s")
    wid = c * NS + s

    def fill_z(i, carry):
        for k in range(D // 16):
            rows0[i, pl.ds(k * 16, 16)] = jnp.zeros((16,), jnp.float32)
        return carry

    lax.fori_loop(0, K, fill_z, 0)

    base = s * ROWS_PER_TILE

    def zero_acc(t, carry):
        pltpu.sync_copy(rows0, acc.at[pl.ds(base + t * K, K)])
        return carry

    lax.fori_loop(0, ROWS_PER_TILE // K, zero_acc, 0)
    plsc.subcore_barrier()

    bufs = (rows0, rows1, rows2, rows3)
    gsems = (gsem0, gsem1, ssem0, ssem1)

    def do_chunk(cb, nb):
        pltpu.sync_copy(src_r.at[wid].at[pl.ds(cb, nb)], srcw.at[pl.ds(0, nb)])
        pltpu.sync_copy(dst_r.at[wid].at[pl.ds(cb, nb)], dstw.at[pl.ds(0, nb)])

        gd = [None] * nb

        def gather(b):
            gd[b] = pltpu.async_copy(
                g.at[srcw.at[b]], bufs[b % 4], gsems[b % 4])

        for p in range(min(3, nb)):
            gather(p)
        for b in range(nb):
            gd[b].wait()
            pltpu.sync_copy(bufs[b % 4], acc.at[dstw.at[b]], add=True)
            if b + 3 < nb:
                gather(b + 3)

    def chunk(cc, carry):
        do_chunk(cc * CB, CB)
        return carry

    lax.fori_loop(0, NFULL, chunk, 0)
    do_chunk(NFULL * CB, NTAIL)
    plsc.subcore_barrier()
    pltpu.sync_copy(
        acc.at[pl.ds(base, ROWS_PER_TILE)],
        out.at[c].at[pl.ds(base, ROWS_PER_TILE)],
    )


BLK = 2000
GRID = N // BLK


def _norm(dcol):
    return jnp.where(dcol > 0, lax.rsqrt(jnp.maximum(dcol, 1e-12)), 0.0)


def _tck1_body(deg_ref, x_ref, w_ref, o_ref):
    ns = _norm(deg_ref[0])
    o_ref[...] = jnp.dot(x_ref[...] * ns, w_ref[...],
                         preferred_element_type=jnp.float32)


def _tck1(deg, x, w):
    return pl.pallas_call(
        _tck1_body,
        grid=(GRID,),
        in_specs=[
            pl.BlockSpec((1, BLK, 1), lambda i: (0, i, 0)),
            pl.BlockSpec((BLK, D), lambda i: (i, 0)),
            pl.BlockSpec((D, D), lambda i: (0, 0)),
        ],
        out_specs=pl.BlockSpec((BLK, D), lambda i: (i, 0)),
        out_shape=jax.ShapeDtypeStruct((N, D), jnp.float32),
    )(deg, x, w)


def _tck2_body(deg_ref, agg_ref, b_ref, w_ref, h1_ref, g2_ref):
    nd = _norm(deg_ref[1])
    ns = _norm(deg_ref[0])
    a = agg_ref[0] + agg_ref[1]
    h1 = jnp.maximum(a * nd + b_ref[...], 0.0)
    h1_ref[...] = h1
    g2_ref[...] = jnp.dot(h1 * ns, w_ref[...],
                          preferred_element_type=jnp.float32)


def _tck2(deg, agg, b, w):
    return pl.pallas_call(
        _tck2_body,
        grid=(GRID,),
        in_specs=[
            pl.BlockSpec((2, BLK, 1), lambda i: (0, i, 0)),
            pl.BlockSpec((2, BLK, D), lambda i: (0, i, 0)),
            pl.BlockSpec((1, D), lambda i: (0, 0)),
            pl.BlockSpec((D, D), lambda i: (0, 0)),
        ],
        out_specs=[
            pl.BlockSpec((BLK, D), lambda i: (i, 0)),
            pl.BlockSpec((BLK, D), lambda i: (i, 0)),
        ],
        out_shape=[
            jax.ShapeDtypeStruct((N, D), jnp.float32),
            jax.ShapeDtypeStruct((N, D), jnp.float32),
        ],
    )(deg, agg, b, w)


def _tck3_body(deg_ref, agg_ref, b_ref, o_ref):
    nd = _norm(deg_ref[1])
    a = agg_ref[0] + agg_ref[1]
    o_ref[...] = a * nd + b_ref[...]


def _tck3(deg, agg, b):
    return pl.pallas_call(
        _tck3_body,
        grid=(GRID,),
        in_specs=[
            pl.BlockSpec((2, BLK, 1), lambda i: (0, i, 0)),
            pl.BlockSpec((2, BLK, D), lambda i: (0, i, 0)),
            pl.BlockSpec((1, D), lambda i: (0, 0)),
        ],
        out_specs=pl.BlockSpec((BLK, D), lambda i: (i, 0)),
        out_shape=jax.ShapeDtypeStruct((N, D), jnp.float32),
    )(deg, agg, b)


def kernel(x, edge_index, W1, b1, W2, b2):
    eidx_deg = edge_index.reshape(2, NS, EPT_DEG)
    src_r = edge_index[0].reshape(NW, NB_AGG, K)
    dst_r = edge_index[1].reshape(NW, NB_AGG, K)
    deg = _deg_kernel(eidx_deg)[:, :, None]
    g1 = _tck1(deg, x, W1)
    agg1 = _agg_kernel(g1, src_r, dst_r)
    h1, g2 = _tck2(deg, agg1, b1.reshape(1, D), W2)
    agg2 = _agg_kernel(g2, src_r, dst_r)
    h2 = _tck3(deg, agg2, b2.reshape(1, D))
    return (h2, h1)
